# baseline (speedup 1.0000x reference)
import functools

import jax
import jax.numpy as jnp
from jax import lax
from jax.experimental import pallas as pl
from jax.experimental.pallas import tpu as pltpu
from jax.experimental.pallas import tpu_sc as plsc

N = 10000
E = 320000
D = 128
_HEADS = (4, 4, 1)

NC = 2
NS = 16
L = 16
NW = NC * NS
EPW = E // NW
G = 16
NG = EPW // G
N_PAD = 10240
ROWS_PER_TILE = N_PAD // NS
DEN_W = 4
DR = 512
SR = 16

_HIGH = lax.Precision.HIGHEST


@functools.cache
def _make_sc_edge_kernel(H):
    VPH = 8 // H

    mesh = plsc.VectorSubcoreMesh(
        core_axis_name="c", subcore_axis_name="s", num_cores=NC, num_subcores=NS
    )

    @functools.partial(
        pl.kernel,
        out_type=[
            jax.ShapeDtypeStruct((NC, N_PAD, D), jnp.float32),
            jax.ShapeDtypeStruct((NC, DR, D), jnp.float32),
        ],
        mesh=mesh,
        compiler_params=pltpu.CompilerParams(needs_layout_passes=False),
        scratch_types=[
            pltpu.VMEM((EPW,), jnp.int32),
            pltpu.VMEM((EPW,), jnp.int32),
            pltpu.VMEM((EPW,), jnp.float32),
            pltpu.VMEM((D,), jnp.float32),
            pltpu.VMEM((D,), jnp.float32),
            pltpu.VMEM((G, D), jnp.float32),
            pltpu.VMEM((G, D), jnp.float32),
            pltpu.VMEM((G, D), jnp.float32),
            pltpu.VMEM((G, D), jnp.float32),
            pltpu.VMEM((G, D), jnp.float32),
            pltpu.VMEM((SR, D), jnp.float32),
            pltpu.VMEM_SHARED((N_PAD, D), jnp.float32),
            pltpu.VMEM_SHARED((DR, D), jnp.float32),
            pltpu.SemaphoreType.DMA,
            pltpu.SemaphoreType.DMA,
            pltpu.SemaphoreType.DMA,
            pltpu.SemaphoreType.DMA,
        ],
    )
    def sc_edge(xl_h, xr_h, src_h, dst_h, ea_h, att_h, we_h, msg_out, den_out,
                src_v, dst_v, ea_v, att_v, we_v,
                gl_a, gr_a, gl_b, gr_b, den_st, stage_v, acc_sh, den_sh,
                sem_gl_a, sem_gr_a, sem_gl_b, sem_gr_b):
        cid = lax.axis_index("c")
        sid = lax.axis_index("s")
        wid = cid * NS + sid
        base = wid * EPW

        pltpu.sync_copy(src_h.at[pl.ds(base, EPW)], src_v)
        pltpu.sync_copy(dst_h.at[pl.ds(base, EPW)], dst_v)
        pltpu.sync_copy(ea_h.at[pl.ds(base, EPW)], ea_v)
        pltpu.sync_copy(att_h, att_v)
        pltpu.sync_copy(we_h, we_v)

        zvec = jnp.zeros((L,), jnp.float32)

        for r in range(SR):
            for v in range(D // L):
                stage_v[r, pl.ds(L * v, L)] = zvec
                den_st[r, pl.ds(L * v, L)] = zvec
        for k in range(ROWS_PER_TILE // SR):
            pltpu.sync_copy(
                stage_v, acc_sh.at[pl.ds(sid * ROWS_PER_TILE + k * SR, SR)]
            )
        for k in range(DR // NS // SR):
            pltpu.sync_copy(
                stage_v, den_sh.at[pl.ds(sid * (DR // NS) + k * SR, SR)]
            )
        plsc.subcore_barrier()

        attr = [att_v[pl.ds(L * v, L)] for v in range(8)]
        wer = [we_v[pl.ds(L * v, L)] for v in range(8)]
        lane = lax.iota(jnp.int32, L)

        def lane_sum(v):
            return jnp.full((L,), plsc.cumsum(v)[L - 1], jnp.float32)

        def issue(g, gl_buf, gr_buf, sem_gl, sem_gr):
            off = g * G
            sidx = src_v[pl.ds(off, G)]
            didx = dst_v[pl.ds(off, G)]
            pltpu.async_copy(xl_h.at[sidx], gl_buf, sem_gl)
            pltpu.async_copy(xr_h.at[didx], gr_buf, sem_gr)

        def wait(gl_buf, gr_buf, sem_gl, sem_gr):
            pltpu.make_async_copy(xl_h.at[pl.ds(0, G)], gl_buf, sem_gl).wait()
            pltpu.make_async_copy(xr_h.at[pl.ds(0, G)], gr_buf, sem_gr).wait()

        def compute(off, gl_buf, gr_buf):
            av = ea_v[pl.ds(off, G)]

            UNR = 4

            def _edge_body(it, ex_t):
                ex_t = list(ex_t)
                for k in range(UNR):
                    e = it * UNR + k
                    a = lane_sum(jnp.where(lane == e, av, zvec))
                    for h in range(H):
                        sacc = None
                        for j in range(VPH):
                            vi = h * VPH + j
                            m = (gl_buf[e, pl.ds(L * vi, L)]
                                 + gr_buf[e, pl.ds(L * vi, L)] + a * wer[vi])
                            m = jnp.maximum(m, 0.2 * m)
                            tv = m * attr[vi]
                            sacc = tv if sacc is None else sacc + tv
                        exv = jnp.exp(lane_sum(sacc))
                        for j in range(VPH):
                            vi = h * VPH + j
                            gr_buf[e, pl.ds(L * vi, L)] = (
                                gl_buf[e, pl.ds(L * vi, L)] * exv
                            )
                        ex_t[h] = jnp.where(lane == e, exv, ex_t[h])
                return tuple(ex_t)

            ex_t = lax.fori_loop(0, G // UNR, _edge_body, (zvec,) * H)
            didx = dst_v[pl.ds(off, G)]
            pltpu.sync_copy(gr_buf, acc_sh.at[didx], add=True)
            col = (didx & 31) * DEN_W
            for h in range(H):
                plsc.store_scatter(den_st, [lane, col + h], ex_t[h])
            pltpu.sync_copy(den_st, den_sh.at[didx >> 5], add=True)
            for h in range(H):
                plsc.store_scatter(den_st, [lane, col + h], zvec)

        issue(0, gl_a, gr_a, sem_gl_a, sem_gr_a)
        issue(1, gl_b, gr_b, sem_gl_b, sem_gr_b)

        HALF = (NG - 1) // 2

        def _loop_body(i, carry):
            wait(gl_a, gr_a, sem_gl_a, sem_gr_a)
            compute(2 * i * G, gl_a, gr_a)
            issue(2 * i + 2, gl_a, gr_a, sem_gl_a, sem_gr_a)
            wait(gl_b, gr_b, sem_gl_b, sem_gr_b)
            compute((2 * i + 1) * G, gl_b, gr_b)

            @pl.when(i < HALF - 1)
            def _():
                issue(2 * i + 3, gl_b, gr_b, sem_gl_b, sem_gr_b)

            return carry

        lax.fori_loop(0, HALF, _loop_body, 0)
        wait(gl_a, gr_a, sem_gl_a, sem_gr_a)
        compute((NG - 1) * G, gl_a, gr_a)

        plsc.subcore_barrier()
        for k in range(ROWS_PER_TILE // SR):
            row0 = sid * ROWS_PER_TILE + k * SR
            pltpu.sync_copy(acc_sh.at[pl.ds(row0, SR)], stage_v)
            pltpu.sync_copy(stage_v, msg_out.at[cid].at[pl.ds(row0, SR)])
        for k in range(DR // NS // SR):
            row0 = sid * (DR // NS) + k * SR
            pltpu.sync_copy(den_sh.at[pl.ds(row0, SR)], stage_v)
            pltpu.sync_copy(stage_v, den_out.at[cid].at[pl.ds(row0, SR)])

    return sc_edge


BR = 1000
GRID = N // BR


def _proj_body(x_ref, wl_ref, wr_ref, xl_ref, xr_ref):
    xb = x_ref[...]
    xl_ref[...] = jnp.dot(xb, wl_ref[...], precision=_HIGH)
    xr_ref[...] = jnp.dot(xb, wr_ref[...], precision=_HIGH)


def _tc_proj(x, wl, wr):
    return pl.pallas_call(
        _proj_body,
        grid=(GRID,),
        in_specs=[
            pl.BlockSpec((BR, D), lambda i: (i, 0)),
            pl.BlockSpec((D, D), lambda i: (0, 0)),
            pl.BlockSpec((D, D), lambda i: (0, 0)),
        ],
        out_specs=[
            pl.BlockSpec((BR, D), lambda i: (i, 0)),
            pl.BlockSpec((BR, D), lambda i: (i, 0)),
        ],
        out_shape=[
            jax.ShapeDtypeStruct((N, D), jnp.float32),
            jax.ShapeDtypeStruct((N, D), jnp.float32),
        ],
    )(x, wl, wr)


def _gat_out(msg0, msg1, den_p, bias, H):
    msg = msg0 + msg1
    den = jnp.sum(den_p, axis=0)
    ch = lax.broadcasted_iota(jnp.int32, (DEN_W, D), 1) // (D // H)
    row = lax.broadcasted_iota(jnp.int32, (DEN_W, D), 0)
    sel = jnp.where(ch == row, 1.0, 0.0)
    den_full = jnp.dot(den, sel, precision=_HIGH)
    return msg / (den_full + 1e-16) + bias


def _ln(y, gamma, beta):
    mu = jnp.mean(y, axis=-1, keepdims=True)
    var = jnp.mean((y - mu) ** 2, axis=-1, keepdims=True)
    return (y - mu) / jnp.sqrt(var + 1e-5) * gamma + beta


def _make_finalize_mid(H):
    def body(a0_ref, a1_ref, dp_ref, x_ref, b_ref, g_ref, be_ref,
             wl_ref, wr_ref, xn_ref, xl_ref, xr_ref):
        y = _gat_out(a0_ref[...], a1_ref[...], dp_ref[...], b_ref[...], H)
        y = _ln(y, g_ref[...], be_ref[...])
        y = jax.nn.gelu(y)
        xn = x_ref[...] + y
        xn_ref[...] = xn
        xl_ref[...] = jnp.dot(xn, wl_ref[...], precision=_HIGH)
        xr_ref[...] = jnp.dot(xn, wr_ref[...], precision=_HIGH)

    return pl.pallas_call(
        body,
        grid=(GRID,),
        in_specs=[
            pl.BlockSpec((BR, D), lambda i: (i, 0)),
            pl.BlockSpec((BR, D), lambda i: (i, 0)),
            pl.BlockSpec((NC, BR, DEN_W), lambda i: (0, i, 0)),
            pl.BlockSpec((BR, D), lambda i: (i, 0)),
            pl.BlockSpec((1, D), lambda i: (0, 0)),
            pl.BlockSpec((1, D), lambda i: (0, 0)),
            pl.BlockSpec((1, D), lambda i: (0, 0)),
            pl.BlockSpec((D, D), lambda i: (0, 0)),
            pl.BlockSpec((D, D), lambda i: (0, 0)),
        ],
        out_specs=[
            pl.BlockSpec((BR, D), lambda i: (i, 0)),
            pl.BlockSpec((BR, D), lambda i: (i, 0)),
            pl.BlockSpec((BR, D), lambda i: (i, 0)),
        ],
        out_shape=[
            jax.ShapeDtypeStruct((N, D), jnp.float32),
            jax.ShapeDtypeStruct((N, D), jnp.float32),
            jax.ShapeDtypeStruct((N, D), jnp.float32),
        ],
    )


def _make_finalize_last(H):
    def body(a0_ref, a1_ref, dp_ref, x_ref, b_ref, g_ref, be_ref, xn_ref):
        y = _gat_out(a0_ref[...], a1_ref[...], dp_ref[...], b_ref[...], H)
        y = _ln(y, g_ref[...], be_ref[...])
        xn_ref[...] = x_ref[...] + y

    return pl.pallas_call(
        body,
        grid=(GRID,),
        in_specs=[
            pl.BlockSpec((BR, D), lambda i: (i, 0)),
            pl.BlockSpec((BR, D), lambda i: (i, 0)),
            pl.BlockSpec((NC, BR, DEN_W), lambda i: (0, i, 0)),
            pl.BlockSpec((BR, D), lambda i: (i, 0)),
            pl.BlockSpec((1, D), lambda i: (0, 0)),
            pl.BlockSpec((1, D), lambda i: (0, 0)),
            pl.BlockSpec((1, D), lambda i: (0, 0)),
        ],
        out_specs=pl.BlockSpec((BR, D), lambda i: (i, 0)),
        out_shape=jax.ShapeDtypeStruct((N, D), jnp.float32),
    )


def kernel(x, edge_index, edge_attr, batch, params):
    src = edge_index[0].astype(jnp.int32)
    dst = edge_index[1].astype(jnp.int32)
    ea = edge_attr[:, 0]

    xl, xr = _tc_proj(x, params[0]["Wl"], params[0]["Wr"])
    for i in range(3):
        H = _HEADS[i]
        p = params[i]
        att = p["att"].reshape(-1)
        we = p["We"].reshape(-1)
        msg, den = _make_sc_edge_kernel(H)(xl, xr, src, dst, ea, att, we)
        den = den.reshape(NC, DR * D // DEN_W, DEN_W)
        bias = p["b"].reshape(1, D)
        gamma = p["gamma"].reshape(1, D)
        beta = p["beta"].reshape(1, D)
        if i < 2:
            x, xl, xr = _make_finalize_mid(H)(
                msg[0], msg[1], den, x, bias, gamma, beta,
                params[i + 1]["Wl"], params[i + 1]["Wr"],
            )
        else:
            x = _make_finalize_last(H)(msg[0], msg[1], den, x, bias, gamma, beta)
    return x[None]


# speedup vs baseline: 1.3938x; 1.3938x over previous
import functools

import jax
import jax.numpy as jnp
from jax import lax
from jax.experimental import pallas as pl
from jax.experimental.pallas import tpu as pltpu
from jax.experimental.pallas import tpu_sc as plsc

N = 10000
E = 320000
D = 128
_HEADS = (4, 4, 1)

NC = 2
NS = 16
L = 16
NW = NC * NS
EPW = E // NW
G = 16
NG = EPW // G
N_PAD = 10240
ROWS_PER_TILE = N_PAD // NS
DEN_W = 4
DR = 512
SR = 16

_HIGH = lax.Precision.HIGHEST


@functools.cache
def _make_sc_edge_kernel(H):
    VPH = 8 // H

    mesh = plsc.VectorSubcoreMesh(
        core_axis_name="c", subcore_axis_name="s", num_cores=NC, num_subcores=NS
    )

    @functools.partial(
        pl.kernel,
        out_type=[
            jax.ShapeDtypeStruct((NC, N_PAD, D), jnp.float32),
            jax.ShapeDtypeStruct((NC, DR, D), jnp.float32),
        ],
        mesh=mesh,
        compiler_params=pltpu.CompilerParams(needs_layout_passes=False),
        scratch_types=[
            pltpu.VMEM((EPW,), jnp.int32),
            pltpu.VMEM((EPW,), jnp.int32),
            pltpu.VMEM((EPW,), jnp.float32),
            pltpu.VMEM((D,), jnp.float32),
            pltpu.VMEM((D,), jnp.float32),
            pltpu.VMEM((G, D), jnp.float32),
            pltpu.VMEM((G, D), jnp.float32),
            pltpu.VMEM((G, D), jnp.float32),
            pltpu.VMEM((G, D), jnp.float32),
            pltpu.VMEM((G, D), jnp.float32),
            pltpu.VMEM((SR, D), jnp.float32),
            pltpu.VMEM_SHARED((N_PAD, D), jnp.float32),
            pltpu.VMEM_SHARED((DR, D), jnp.float32),
            pltpu.SemaphoreType.DMA,
            pltpu.SemaphoreType.DMA,
            pltpu.SemaphoreType.DMA,
            pltpu.SemaphoreType.DMA,
        ],
    )
    def sc_edge(xl_h, xr_h, src_h, dst_h, ea_h, att_h, we_h, msg_out, den_out,
                src_v, dst_v, ea_v, att_v, we_v,
                gl_a, gr_a, gl_b, gr_b, den_st, stage_v, acc_sh, den_sh,
                sem_gl_a, sem_gr_a, sem_gl_b, sem_gr_b):
        cid = lax.axis_index("c")
        sid = lax.axis_index("s")
        wid = cid * NS + sid
        base = wid * EPW

        pltpu.sync_copy(src_h.at[pl.ds(base, EPW)], src_v)
        pltpu.sync_copy(dst_h.at[pl.ds(base, EPW)], dst_v)
        pltpu.sync_copy(ea_h.at[pl.ds(base, EPW)], ea_v)
        pltpu.sync_copy(att_h, att_v)
        pltpu.sync_copy(we_h, we_v)

        zvec = jnp.zeros((L,), jnp.float32)

        for r in range(SR):
            for v in range(D // L):
                stage_v[r, pl.ds(L * v, L)] = zvec
                den_st[r, pl.ds(L * v, L)] = zvec
        for k in range(ROWS_PER_TILE // SR):
            pltpu.sync_copy(
                stage_v, acc_sh.at[pl.ds(sid * ROWS_PER_TILE + k * SR, SR)]
            )
        for k in range(DR // NS // SR):
            pltpu.sync_copy(
                stage_v, den_sh.at[pl.ds(sid * (DR // NS) + k * SR, SR)]
            )
        plsc.subcore_barrier()

        attr = [att_v[pl.ds(L * v, L)] for v in range(8)]
        wer = [we_v[pl.ds(L * v, L)] for v in range(8)]
        lane = lax.iota(jnp.int32, L)

        def lane_sum(v):
            return jnp.full((L,), plsc.cumsum(v)[L - 1], jnp.float32)

        def issue(g, gl_buf, gr_buf, sem_gl, sem_gr):
            off = g * G
            sidx = src_v[pl.ds(off, G)]
            didx = dst_v[pl.ds(off, G)]
            pltpu.async_copy(xl_h.at[sidx], gl_buf, sem_gl)
            pltpu.async_copy(xr_h.at[didx], gr_buf, sem_gr)

        def wait(gl_buf, gr_buf, sem_gl, sem_gr):
            pltpu.make_async_copy(xl_h.at[pl.ds(0, G)], gl_buf, sem_gl).wait()
            pltpu.make_async_copy(xr_h.at[pl.ds(0, G)], gr_buf, sem_gr).wait()

        def compute(off, gl_buf, gr_buf):
            ex_t = [zvec for _ in range(H)]
            for e in range(G):
                a = plsc.load_gather(ea_v, [jnp.full((L,), off + e, jnp.int32)])
                for h in range(H):
                    sacc = None
                    for j in range(VPH):
                        vi = h * VPH + j
                        m = (gl_buf[e, pl.ds(L * vi, L)]
                             + gr_buf[e, pl.ds(L * vi, L)] + a * wer[vi])
                        m = jnp.maximum(m, 0.2 * m)
                        tv = m * attr[vi]
                        sacc = tv if sacc is None else sacc + tv
                    exv = jnp.exp(lane_sum(sacc))
                    for j in range(VPH):
                        vi = h * VPH + j
                        gr_buf[e, pl.ds(L * vi, L)] = (
                            gl_buf[e, pl.ds(L * vi, L)] * exv
                        )
                    ex_t[h] = jnp.where(lane == e, exv, ex_t[h])
            didx = dst_v[pl.ds(off, G)]
            pltpu.sync_copy(gr_buf, acc_sh.at[didx], add=True)
            col = (didx & 31) * DEN_W
            for h in range(H):
                plsc.store_scatter(den_st, [lane, col + h], ex_t[h])
            pltpu.sync_copy(den_st, den_sh.at[didx >> 5], add=True)
            for h in range(H):
                plsc.store_scatter(den_st, [lane, col + h], zvec)

        issue(0, gl_a, gr_a, sem_gl_a, sem_gr_a)
        issue(1, gl_b, gr_b, sem_gl_b, sem_gr_b)

        HALF = (NG - 1) // 2

        def _loop_body(i, carry):
            wait(gl_a, gr_a, sem_gl_a, sem_gr_a)
            compute(2 * i * G, gl_a, gr_a)
            issue(2 * i + 2, gl_a, gr_a, sem_gl_a, sem_gr_a)
            wait(gl_b, gr_b, sem_gl_b, sem_gr_b)
            compute((2 * i + 1) * G, gl_b, gr_b)

            @pl.when(i < HALF - 1)
            def _():
                issue(2 * i + 3, gl_b, gr_b, sem_gl_b, sem_gr_b)

            return carry

        lax.fori_loop(0, HALF, _loop_body, 0)
        wait(gl_a, gr_a, sem_gl_a, sem_gr_a)
        compute((NG - 1) * G, gl_a, gr_a)

        plsc.subcore_barrier()
        for k in range(ROWS_PER_TILE // SR):
            row0 = sid * ROWS_PER_TILE + k * SR
            pltpu.sync_copy(acc_sh.at[pl.ds(row0, SR)], stage_v)
            pltpu.sync_copy(stage_v, msg_out.at[cid].at[pl.ds(row0, SR)])
        for k in range(DR // NS // SR):
            row0 = sid * (DR // NS) + k * SR
            pltpu.sync_copy(den_sh.at[pl.ds(row0, SR)], stage_v)
            pltpu.sync_copy(stage_v, den_out.at[cid].at[pl.ds(row0, SR)])

    return sc_edge


BR = 1000
GRID = N // BR


def _proj_body(x_ref, wl_ref, wr_ref, xl_ref, xr_ref):
    xb = x_ref[...]
    xl_ref[...] = jnp.dot(xb, wl_ref[...], precision=_HIGH)
    xr_ref[...] = jnp.dot(xb, wr_ref[...], precision=_HIGH)


def _tc_proj(x, wl, wr):
    return pl.pallas_call(
        _proj_body,
        grid=(GRID,),
        in_specs=[
            pl.BlockSpec((BR, D), lambda i: (i, 0)),
            pl.BlockSpec((D, D), lambda i: (0, 0)),
            pl.BlockSpec((D, D), lambda i: (0, 0)),
        ],
        out_specs=[
            pl.BlockSpec((BR, D), lambda i: (i, 0)),
            pl.BlockSpec((BR, D), lambda i: (i, 0)),
        ],
        out_shape=[
            jax.ShapeDtypeStruct((N, D), jnp.float32),
            jax.ShapeDtypeStruct((N, D), jnp.float32),
        ],
    )(x, wl, wr)


def _gat_out(msg0, msg1, den_p, bias, H):
    msg = msg0 + msg1
    den = jnp.sum(den_p, axis=0)
    ch = lax.broadcasted_iota(jnp.int32, (DEN_W, D), 1) // (D // H)
    row = lax.broadcasted_iota(jnp.int32, (DEN_W, D), 0)
    sel = jnp.where(ch == row, 1.0, 0.0)
    den_full = jnp.dot(den, sel, precision=_HIGH)
    return msg / (den_full + 1e-16) + bias


def _ln(y, gamma, beta):
    mu = jnp.mean(y, axis=-1, keepdims=True)
    var = jnp.mean((y - mu) ** 2, axis=-1, keepdims=True)
    return (y - mu) / jnp.sqrt(var + 1e-5) * gamma + beta


def _make_finalize_mid(H):
    def body(a0_ref, a1_ref, dp_ref, x_ref, b_ref, g_ref, be_ref,
             wl_ref, wr_ref, xn_ref, xl_ref, xr_ref):
        y = _gat_out(a0_ref[...], a1_ref[...], dp_ref[...], b_ref[...], H)
        y = _ln(y, g_ref[...], be_ref[...])
        y = jax.nn.gelu(y)
        xn = x_ref[...] + y
        xn_ref[...] = xn
        xl_ref[...] = jnp.dot(xn, wl_ref[...], precision=_HIGH)
        xr_ref[...] = jnp.dot(xn, wr_ref[...], precision=_HIGH)

    return pl.pallas_call(
        body,
        grid=(GRID,),
        in_specs=[
            pl.BlockSpec((BR, D), lambda i: (i, 0)),
            pl.BlockSpec((BR, D), lambda i: (i, 0)),
            pl.BlockSpec((NC, BR, DEN_W), lambda i: (0, i, 0)),
            pl.BlockSpec((BR, D), lambda i: (i, 0)),
            pl.BlockSpec((1, D), lambda i: (0, 0)),
            pl.BlockSpec((1, D), lambda i: (0, 0)),
            pl.BlockSpec((1, D), lambda i: (0, 0)),
            pl.BlockSpec((D, D), lambda i: (0, 0)),
            pl.BlockSpec((D, D), lambda i: (0, 0)),
        ],
        out_specs=[
            pl.BlockSpec((BR, D), lambda i: (i, 0)),
            pl.BlockSpec((BR, D), lambda i: (i, 0)),
            pl.BlockSpec((BR, D), lambda i: (i, 0)),
        ],
        out_shape=[
            jax.ShapeDtypeStruct((N, D), jnp.float32),
            jax.ShapeDtypeStruct((N, D), jnp.float32),
            jax.ShapeDtypeStruct((N, D), jnp.float32),
        ],
    )


def _make_finalize_last(H):
    def body(a0_ref, a1_ref, dp_ref, x_ref, b_ref, g_ref, be_ref, xn_ref):
        y = _gat_out(a0_ref[...], a1_ref[...], dp_ref[...], b_ref[...], H)
        y = _ln(y, g_ref[...], be_ref[...])
        xn_ref[...] = x_ref[...] + y

    return pl.pallas_call(
        body,
        grid=(GRID,),
        in_specs=[
            pl.BlockSpec((BR, D), lambda i: (i, 0)),
            pl.BlockSpec((BR, D), lambda i: (i, 0)),
            pl.BlockSpec((NC, BR, DEN_W), lambda i: (0, i, 0)),
            pl.BlockSpec((BR, D), lambda i: (i, 0)),
            pl.BlockSpec((1, D), lambda i: (0, 0)),
            pl.BlockSpec((1, D), lambda i: (0, 0)),
            pl.BlockSpec((1, D), lambda i: (0, 0)),
        ],
        out_specs=pl.BlockSpec((BR, D), lambda i: (i, 0)),
        out_shape=jax.ShapeDtypeStruct((N, D), jnp.float32),
    )


def kernel(x, edge_index, edge_attr, batch, params):
    src = edge_index[0].astype(jnp.int32)
    dst = edge_index[1].astype(jnp.int32)
    ea = edge_attr[:, 0]

    xl, xr = _tc_proj(x, params[0]["Wl"], params[0]["Wr"])
    for i in range(3):
        H = _HEADS[i]
        p = params[i]
        att = p["att"].reshape(-1)
        we = p["We"].reshape(-1)
        msg, den = _make_sc_edge_kernel(H)(xl, xr, src, dst, ea, att, we)
        den = den.reshape(NC, DR * D // DEN_W, DEN_W)
        bias = p["b"].reshape(1, D)
        gamma = p["gamma"].reshape(1, D)
        beta = p["beta"].reshape(1, D)
        if i < 2:
            x, xl, xr = _make_finalize_mid(H)(
                msg[0], msg[1], den, x, bias, gamma, beta,
                params[i + 1]["Wl"], params[i + 1]["Wr"],
            )
        else:
            x = _make_finalize_last(H)(msg[0], msg[1], den, x, bias, gamma, beta)
    return x[None]


# speedup vs baseline: 1.4382x; 1.0318x over previous
import functools

import jax
import jax.numpy as jnp
from jax import lax
from jax.experimental import pallas as pl
from jax.experimental.pallas import tpu as pltpu
from jax.experimental.pallas import tpu_sc as plsc

N = 10000
E = 320000
D = 128
_HEADS = (4, 4, 1)

NC = 2
NS = 16
L = 16
NW = NC * NS
EPW = E // NW
G = 16
EPW_P = EPW + 2 * G
NG_P = EPW_P // G
NITER = NG_P // 3
N_PAD = 10240
ROWS_PER_TILE = N_PAD // NS
DEN_W = 4
DR = 512
SR = 16

_HIGH = lax.Precision.HIGHEST


@functools.cache
def _make_sc_edge_kernel(H):
    VPH = 8 // H

    mesh = plsc.VectorSubcoreMesh(
        core_axis_name="c", subcore_axis_name="s", num_cores=NC, num_subcores=NS
    )

    @functools.partial(
        pl.kernel,
        out_type=[
            jax.ShapeDtypeStruct((NC, N_PAD, D), jnp.float32),
            jax.ShapeDtypeStruct((NC, DR, D), jnp.float32),
        ],
        mesh=mesh,
        compiler_params=pltpu.CompilerParams(needs_layout_passes=False),
        scratch_types=[
            pltpu.VMEM((EPW_P,), jnp.int32),
            pltpu.VMEM((EPW_P,), jnp.int32),
            pltpu.VMEM((D,), jnp.float32),
            pltpu.VMEM((D,), jnp.float32),
            pltpu.VMEM((G, D), jnp.float32),
            pltpu.VMEM((G, D), jnp.float32),
            pltpu.VMEM((G,), jnp.float32),
            pltpu.VMEM((G, D), jnp.float32),
            pltpu.VMEM((G, D), jnp.float32),
            pltpu.VMEM((G, D), jnp.float32),
            pltpu.VMEM((G,), jnp.float32),
            pltpu.VMEM((G, D), jnp.float32),
            pltpu.VMEM((G, D), jnp.float32),
            pltpu.VMEM((G, D), jnp.float32),
            pltpu.VMEM((G,), jnp.float32),
            pltpu.VMEM((G, D), jnp.float32),
            pltpu.VMEM((SR, D), jnp.float32),
            pltpu.VMEM_SHARED((N_PAD, D), jnp.float32),
            pltpu.VMEM_SHARED((DR, D), jnp.float32),
            pltpu.SemaphoreType.DMA,
            pltpu.SemaphoreType.DMA,
            pltpu.SemaphoreType.DMA,
            pltpu.SemaphoreType.DMA,
            pltpu.SemaphoreType.DMA,
            pltpu.SemaphoreType.DMA,
        ],
    )
    def sc_edge(xl_h, xr_h, src_h, dst_h, ea_h, att_h, we_h, msg_out, den_out,
                src_v, dst_v, att_v, we_v,
                gl_p, gr_p, ea_p, den_p,
                gl_q, gr_q, ea_q, den_q,
                gl_r, gr_r, ea_r, den_r,
                stage_v, acc_sh, den_sh,
                sem_g_p, sem_g_q, sem_g_r, sem_s_p, sem_s_q, sem_s_r):
        cid = lax.axis_index("c")
        sid = lax.axis_index("s")
        wid = cid * NS + sid

        pltpu.sync_copy(src_h.at[wid], src_v)
        pltpu.sync_copy(dst_h.at[wid], dst_v)
        pltpu.sync_copy(att_h, att_v)
        pltpu.sync_copy(we_h, we_v)

        zvec = jnp.zeros((L,), jnp.float32)

        for r in range(SR):
            for v in range(D // L):
                stage_v[r, pl.ds(L * v, L)] = zvec
                den_p[r, pl.ds(L * v, L)] = zvec
                den_q[r, pl.ds(L * v, L)] = zvec
                den_r[r, pl.ds(L * v, L)] = zvec
        for k in range(ROWS_PER_TILE // SR):
            pltpu.sync_copy(
                stage_v, acc_sh.at[pl.ds(sid * ROWS_PER_TILE + k * SR, SR)]
            )
        for k in range(DR // NS // SR):
            pltpu.sync_copy(
                stage_v, den_sh.at[pl.ds(sid * (DR // NS) + k * SR, SR)]
            )
        plsc.subcore_barrier()

        attr = [att_v[pl.ds(L * v, L)] for v in range(8)]
        wer = [we_v[pl.ds(L * v, L)] for v in range(8)]
        lane = lax.iota(jnp.int32, L)

        def lane_sum(v):
            return jnp.full((L,), plsc.cumsum(v)[L - 1], jnp.float32)

        def issue(g, gl_b, gr_b, ea_b, sem):
            off = g * G
            sidx = src_v[pl.ds(off, G)]
            didx = dst_v[pl.ds(off, G)]
            pltpu.async_copy(xl_h.at[sidx], gl_b, sem)
            pltpu.async_copy(xr_h.at[didx], gr_b, sem)
            pltpu.async_copy(ea_h.at[wid, pl.ds(off, G)], ea_b, sem)

        def wait_g(gl_b, gr_b, ea_b, sem):
            pltpu.make_async_copy(xl_h.at[pl.ds(0, G)], gl_b, sem).wait()
            pltpu.make_async_copy(xr_h.at[pl.ds(0, G)], gr_b, sem).wait()
            pltpu.make_async_copy(ea_h.at[0, pl.ds(0, G)], ea_b, sem).wait()

        def scat(g, gr_b, den_b, sem):
            off = g * G
            didx = dst_v[pl.ds(off, G)]
            pltpu.async_copy(gr_b, acc_sh.at[didx], sem, add=True)
            pltpu.async_copy(den_b, den_sh.at[didx >> 5], sem, add=True)

        def wait_s(gr_b, den_b, sem):
            pltpu.make_async_copy(gr_b, acc_sh.at[pl.ds(0, G)], sem).wait()
            pltpu.make_async_copy(den_b, den_sh.at[pl.ds(0, G)], sem).wait()

        def den_clear(g, den_b):
            didx = dst_v[pl.ds(g * G, G)]
            col = (didx & 31) * DEN_W
            for h in range(H):
                plsc.store_scatter(den_b, [lane, col + h], zvec)

        def compute(g, gl_b, gr_b, ea_b, den_b):
            ex_t = [zvec for _ in range(H)]
            for e in range(G):
                a = plsc.load_gather(ea_b, [jnp.full((L,), e, jnp.int32)])
                for h in range(H):
                    sacc = None
                    for j in range(VPH):
                        vi = h * VPH + j
                        m = (gl_b[e, pl.ds(L * vi, L)]
                             + gr_b[e, pl.ds(L * vi, L)] + a * wer[vi])
                        m = jnp.maximum(m, 0.2 * m)
                        tv = m * attr[vi]
                        sacc = tv if sacc is None else sacc + tv
                    exv = jnp.exp(lane_sum(sacc))
                    for j in range(VPH):
                        vi = h * VPH + j
                        gr_b[e, pl.ds(L * vi, L)] = (
                            gl_b[e, pl.ds(L * vi, L)] * exv
                        )
                    ex_t[h] = jnp.where(lane == e, exv, ex_t[h])
            didx = dst_v[pl.ds(g * G, G)]
            col = (didx & 31) * DEN_W
            for h in range(H):
                plsc.store_scatter(den_b, [lane, col + h], ex_t[h])

        issue(0, gl_p, gr_p, ea_p, sem_g_p)
        issue(1, gl_q, gr_q, ea_q, sem_g_q)

        def _body(i, carry):
            g0 = 3 * i

            @pl.when(i > 0)
            def _():
                wait_s(gr_r, den_r, sem_s_r)
                den_clear(g0 - 1, den_r)

            issue(g0 + 2, gl_r, gr_r, ea_r, sem_g_r)

            wait_g(gl_p, gr_p, ea_p, sem_g_p)
            compute(g0, gl_p, gr_p, ea_p, den_p)
            scat(g0, gr_p, den_p, sem_s_p)

            wait_g(gl_q, gr_q, ea_q, sem_g_q)
            compute(g0 + 1, gl_q, gr_q, ea_q, den_q)
            scat(g0 + 1, gr_q, den_q, sem_s_q)

            wait_s(gr_p, den_p, sem_s_p)
            den_clear(g0, den_p)

            @pl.when(g0 + 3 < NG_P)
            def _():
                issue(g0 + 3, gl_p, gr_p, ea_p, sem_g_p)

            wait_g(gl_r, gr_r, ea_r, sem_g_r)
            compute(g0 + 2, gl_r, gr_r, ea_r, den_r)
            scat(g0 + 2, gr_r, den_r, sem_s_r)

            wait_s(gr_q, den_q, sem_s_q)
            den_clear(g0 + 1, den_q)

            @pl.when(g0 + 4 < NG_P)
            def _():
                issue(g0 + 4, gl_q, gr_q, ea_q, sem_g_q)

            return carry

        lax.fori_loop(0, NITER, _body, 0)
        wait_s(gr_r, den_r, sem_s_r)

        plsc.subcore_barrier()
        for k in range(ROWS_PER_TILE // SR):
            row0 = sid * ROWS_PER_TILE + k * SR
            pltpu.sync_copy(acc_sh.at[pl.ds(row0, SR)], stage_v)
            pltpu.sync_copy(stage_v, msg_out.at[cid].at[pl.ds(row0, SR)])
        for k in range(DR // NS // SR):
            row0 = sid * (DR // NS) + k * SR
            pltpu.sync_copy(den_sh.at[pl.ds(row0, SR)], stage_v)
            pltpu.sync_copy(stage_v, den_out.at[cid].at[pl.ds(row0, SR)])

    return sc_edge


BR = 1024
GRID = N_PAD // BR


def _proj_body(x_ref, wl_ref, wr_ref, xl_ref, xr_ref):
    xb = x_ref[...]
    xl_ref[...] = jnp.dot(xb, wl_ref[...], precision=_HIGH)
    xr_ref[...] = jnp.dot(xb, wr_ref[...], precision=_HIGH)


def _tc_proj(x, wl, wr):
    return pl.pallas_call(
        _proj_body,
        grid=(GRID,),
        in_specs=[
            pl.BlockSpec((BR, D), lambda i: (i, 0)),
            pl.BlockSpec((D, D), lambda i: (0, 0)),
            pl.BlockSpec((D, D), lambda i: (0, 0)),
        ],
        out_specs=[
            pl.BlockSpec((BR, D), lambda i: (i, 0)),
            pl.BlockSpec((BR, D), lambda i: (i, 0)),
        ],
        out_shape=[
            jax.ShapeDtypeStruct((N_PAD, D), jnp.float32),
            jax.ShapeDtypeStruct((N_PAD, D), jnp.float32),
        ],
    )(x, wl, wr)


def _gat_out(msg0, msg1, den_p, bias, H):
    msg = msg0 + msg1
    den = jnp.sum(den_p, axis=0)
    ch = lax.broadcasted_iota(jnp.int32, (DEN_W, D), 1) // (D // H)
    row = lax.broadcasted_iota(jnp.int32, (DEN_W, D), 0)
    sel = jnp.where(ch == row, 1.0, 0.0)
    den_full = jnp.dot(den, sel, precision=_HIGH)
    return msg / (den_full + 1e-16) + bias


def _ln(y, gamma, beta):
    mu = jnp.mean(y, axis=-1, keepdims=True)
    var = jnp.mean((y - mu) ** 2, axis=-1, keepdims=True)
    return (y - mu) / jnp.sqrt(var + 1e-5) * gamma + beta


def _make_finalize_mid(H):
    def body(a0_ref, a1_ref, dp_ref, x_ref, b_ref, g_ref, be_ref,
             wl_ref, wr_ref, xn_ref, xl_ref, xr_ref):
        y = _gat_out(a0_ref[...], a1_ref[...], dp_ref[...], b_ref[...], H)
        y = _ln(y, g_ref[...], be_ref[...])
        y = jax.nn.gelu(y)
        xn = x_ref[...] + y
        xn_ref[...] = xn
        xl_ref[...] = jnp.dot(xn, wl_ref[...], precision=_HIGH)
        xr_ref[...] = jnp.dot(xn, wr_ref[...], precision=_HIGH)

    return pl.pallas_call(
        body,
        grid=(GRID,),
        in_specs=[
            pl.BlockSpec((BR, D), lambda i: (i, 0)),
            pl.BlockSpec((BR, D), lambda i: (i, 0)),
            pl.BlockSpec((NC, BR, DEN_W), lambda i: (0, i, 0)),
            pl.BlockSpec((BR, D), lambda i: (i, 0)),
            pl.BlockSpec((1, D), lambda i: (0, 0)),
            pl.BlockSpec((1, D), lambda i: (0, 0)),
            pl.BlockSpec((1, D), lambda i: (0, 0)),
            pl.BlockSpec((D, D), lambda i: (0, 0)),
            pl.BlockSpec((D, D), lambda i: (0, 0)),
        ],
        out_specs=[
            pl.BlockSpec((BR, D), lambda i: (i, 0)),
            pl.BlockSpec((BR, D), lambda i: (i, 0)),
            pl.BlockSpec((BR, D), lambda i: (i, 0)),
        ],
        out_shape=[
            jax.ShapeDtypeStruct((N_PAD, D), jnp.float32),
            jax.ShapeDtypeStruct((N_PAD, D), jnp.float32),
            jax.ShapeDtypeStruct((N_PAD, D), jnp.float32),
        ],
    )


def _make_finalize_last(H):
    def body(a0_ref, a1_ref, dp_ref, x_ref, b_ref, g_ref, be_ref, xn_ref):
        y = _gat_out(a0_ref[...], a1_ref[...], dp_ref[...], b_ref[...], H)
        y = _ln(y, g_ref[...], be_ref[...])
        xn_ref[...] = x_ref[...] + y

    return pl.pallas_call(
        body,
        grid=(GRID,),
        in_specs=[
            pl.BlockSpec((BR, D), lambda i: (i, 0)),
            pl.BlockSpec((BR, D), lambda i: (i, 0)),
            pl.BlockSpec((NC, BR, DEN_W), lambda i: (0, i, 0)),
            pl.BlockSpec((BR, D), lambda i: (i, 0)),
            pl.BlockSpec((1, D), lambda i: (0, 0)),
            pl.BlockSpec((1, D), lambda i: (0, 0)),
            pl.BlockSpec((1, D), lambda i: (0, 0)),
        ],
        out_specs=pl.BlockSpec((BR, D), lambda i: (i, 0)),
        out_shape=jax.ShapeDtypeStruct((N_PAD, D), jnp.float32),
    )


def kernel(x, edge_index, edge_attr, batch, params):
    src = edge_index[0].astype(jnp.int32)
    dst = edge_index[1].astype(jnp.int32)
    ea = edge_attr[:, 0]

    x = jnp.pad(x, ((0, N_PAD - N), (0, 0)))
    pad_i = jnp.full((NW, EPW_P - EPW), N_PAD - 1, jnp.int32)
    src = jnp.concatenate([src.reshape(NW, EPW), pad_i], axis=1)
    dst = jnp.concatenate([dst.reshape(NW, EPW), pad_i], axis=1)
    ea = jnp.concatenate(
        [ea.reshape(NW, EPW), jnp.zeros((NW, EPW_P - EPW), jnp.float32)],
        axis=1,
    )

    xl, xr = _tc_proj(x, params[0]["Wl"], params[0]["Wr"])
    for i in range(3):
        H = _HEADS[i]
        p = params[i]
        att = p["att"].reshape(-1)
        we = p["We"].reshape(-1)
        msg, den = _make_sc_edge_kernel(H)(xl, xr, src, dst, ea, att, we)
        den = den.reshape(NC, DR * D // DEN_W, DEN_W)
        bias = p["b"].reshape(1, D)
        gamma = p["gamma"].reshape(1, D)
        beta = p["beta"].reshape(1, D)
        if i < 2:
            x, xl, xr = _make_finalize_mid(H)(
                msg[0], msg[1], den, x, bias, gamma, beta,
                params[i + 1]["Wl"], params[i + 1]["Wr"],
            )
        else:
            x = _make_finalize_last(H)(msg[0], msg[1], den, x, bias, gamma, beta)
    return x[:N][None]
